# baseline (speedup 1.0000x reference)
import jax
import jax.numpy as jnp
from jax.experimental import pallas as pl

_K = 16
_B = 32


def _dot(a, b):
    return jax.lax.dot_general(
        a, b, (((1,), (0,)), ((), ())), preferred_element_type=jnp.float32
    )


def _split(x):
    hi = x.astype(jnp.bfloat16)
    lo = (x - hi.astype(jnp.float32)).astype(jnp.bfloat16)
    return jnp.concatenate([hi, lo], axis=0)


def _extract_l1(pos_c, posT_c, w1cT, w1bT, b1c, w2T, b2c):
    P = pos_c.shape[0]
    f32 = jnp.float32
    bf16 = jnp.bfloat16

    d = None
    for a in range(3):
        diff = pos_c[:, a:a + 1] - posT_c[a:a + 1, :]
        sq = diff * diff
        d = sq if d is None else d + sq

    row = jax.lax.broadcasted_iota(jnp.int32, (P, P), 0)
    col = jax.lax.broadcasted_iota(jnp.int32, (P, P), 1)

    u1T = _dot(w1cT, posT_c)
    v1T = _dot(w1bT, posT_c)
    H = v1T.shape[0]
    u1s = _split(u1T)

    work = jnp.where(row == col, jnp.inf, d)
    acc1 = _dot(w2T, jnp.maximum(u1T - v1T + b1c, 0.0))

    sels = []
    for _ in range(1, _K):
        m = jnp.min(work, axis=0, keepdims=True)
        jsel = jnp.min(jnp.where(work == m, row, P), axis=0,
                       keepdims=True)
        sels.append(jsel)
        sel = row == jsel
        gs = _dot(u1s, sel.astype(bf16))
        gT = gs[:H] + gs[H:]
        msgT = jnp.maximum(gT - v1T + b1c, 0.0)
        acc1 = jnp.maximum(acc1, _dot(w2T, msgT))
        work = jnp.where(sel, jnp.inf, work)
    h1T = jnp.maximum(acc1 + b2c, 0.0)
    return h1T, sels


def _layer2_out(h1T, posT_c, sels, w3aT, w3bT, b3c, w4T, b4c, wc, bcr):
    P = h1T.shape[1]
    H = h1T.shape[0]
    f32 = jnp.float32
    bf16 = jnp.bfloat16
    row = jax.lax.broadcasted_iota(jnp.int32, (P, P), 0)

    pvT = _dot(w3bT, posT_c)
    u2T = _dot(w3aT, h1T) + pvT
    u2s = _split(u2T)

    acc2 = _dot(w4T, jnp.maximum(u2T - pvT + b3c, 0.0))
    for t in range(_K - 1):
        gs = _dot(u2s, (row == sels[t]).astype(bf16))
        gT = gs[:H] + gs[H:]
        msgT = jnp.maximum(gT - pvT + b3c, 0.0)
        acc2 = jnp.maximum(acc2, _dot(w4T, msgT))
    h2T = jnp.maximum(acc2 + b4c, 0.0)

    g = jnp.max(h2T, axis=1, keepdims=True)
    out = jax.lax.dot_general(
        g, wc, (((0,), (0,)), ((), ())), preferred_element_type=f32)
    return out + bcr


def _pair_kernel(pos_ref, posT_ref, w1cT_ref, w1bT_ref, b1_ref, w2T_ref,
                 b2_ref, w3aT_ref, w3bT_ref, b3_ref, w4T_ref, b4_ref,
                 wc_ref, bc_ref, out_ref):
    P = pos_ref.shape[0] // 2
    l1_args = (w1cT_ref[...], w1bT_ref[...], b1_ref[...], w2T_ref[...],
               b2_ref[...])
    l2_args = (w3aT_ref[...], w3bT_ref[...], b3_ref[...], w4T_ref[...],
               b4_ref[...], wc_ref[...], bc_ref[...])

    pos_a, pos_b = pos_ref[:P], pos_ref[P:]
    posT_a, posT_b = posT_ref[:, :P], posT_ref[:, P:]

    h1a, sela = _extract_l1(pos_a, posT_a, *l1_args)
    out_a = _layer2_out(h1a, posT_a, sela, *l2_args)
    h1b, selb = _extract_l1(pos_b, posT_b, *l1_args)
    out_b = _layer2_out(h1b, posT_b, selb, *l2_args)

    out = jnp.concatenate([out_a, out_b], axis=0)
    out_ref[...] = out.reshape(out_ref.shape)


def kernel(pos, batch, W1, b1, W2, b2, W3, b3, W4, b4, Wc, bc):
    del batch
    N = pos.shape[0]
    P = N // _B
    H = W2.shape[0]
    NCLS = Wc.shape[1]

    posT = pos.T
    w1cT = (W1[:3] + W1[3:]).T
    w1bT = W1[3:].T
    w3aT = W3[:H].T
    w3bT = W3[H:].T

    full = lambda shape: pl.BlockSpec(shape, lambda b: (0, 0))
    out = pl.pallas_call(
        _pair_kernel,
        grid=(_B // 2,),
        in_specs=[
            pl.BlockSpec((2 * P, 3), lambda b: (b, 0)),
            pl.BlockSpec((3, 2 * P), lambda b: (0, b)),
            full((H, 3)),
            full((H, 3)),
            full((H, 1)),
            full((H, H)),
            full((H, 1)),
            full((H, H)),
            full((H, 3)),
            full((H, 1)),
            full((H, H)),
            full((H, 1)),
            full((H, NCLS)),
            full((1, NCLS)),
        ],
        out_specs=pl.BlockSpec((2, 1, NCLS), lambda b: (b, 0, 0)),
        out_shape=jax.ShapeDtypeStruct((_B, 1, NCLS), jnp.float32),
    )(
        pos, posT, w1cT, w1bT, b1.reshape(H, 1), W2.T, b2.reshape(H, 1),
        w3aT, w3bT, b3.reshape(H, 1), W4.T, b4.reshape(H, 1), Wc,
        bc.reshape(1, NCLS),
    )
    return out.reshape(_B, NCLS)


# speedup vs baseline: 1.2711x; 1.2711x over previous
import functools

import jax
import jax.numpy as jnp
from jax.experimental import pallas as pl
from jax.experimental.pallas import tpu as pltpu

_K = 16
_B = 32


def _dot(a, b):
    return jax.lax.dot_general(
        a, b, (((1,), (0,)), ((), ())), preferred_element_type=jnp.float32
    )


def _cloud_kernel(pos_ref, posT_ref, w1cT_ref, w1bT_ref, b1_ref, w2T_ref,
                  b2_ref, w3aT_ref, w3bT_ref, b3_ref, w4T_ref, b4_ref,
                  wc_ref, bc_ref, out_ref, oh_ref):
    P = pos_ref.shape[0]
    f32 = jnp.float32
    pos_c = pos_ref[...]
    posT_c = posT_ref[...]

    d = None
    for a in range(3):
        diff = pos_c[:, a:a + 1] - posT_c[a:a + 1, :]
        sq = diff * diff
        d = sq if d is None else d + sq

    row = jax.lax.broadcasted_iota(jnp.int32, (P, P), 0).astype(f32)

    bf16 = jnp.bfloat16

    def _split(x):
        hi = x.astype(bf16)
        lo = (x - hi.astype(f32)).astype(bf16)
        return jnp.concatenate([hi, lo], axis=0)

    u1T = _dot(w1cT_ref[...], posT_c)
    v1T = _dot(w1bT_ref[...], posT_c)
    H = v1T.shape[0]
    u1s = _split(u1T)

    b1c = b1_ref[...]
    w2T = w2T_ref[...]

    col = jax.lax.broadcasted_iota(jnp.int32, (P, P), 1).astype(f32)
    diag = row == col
    work = jnp.where(diag, jnp.inf, d)
    msgT = jnp.maximum(u1T - v1T + b1c, 0.0)
    acc1 = _dot(w2T, msgT)

    for t in range(1, _K):
        m = jnp.min(work, axis=0, keepdims=True)
        jsel = jnp.min(jnp.where(work == m, row, float(P)), axis=0,
                       keepdims=True)
        sel = row == jsel
        ohb = sel.astype(bf16)
        oh_ref[t - 1] = ohb
        gs = _dot(u1s, ohb)
        gT = gs[:H] + gs[H:]
        msgT = jnp.maximum(gT - v1T + b1c, 0.0)
        acc1 = jnp.maximum(acc1, _dot(w2T, msgT))
        work = jnp.where(sel, jnp.inf, work)
    h1T = jnp.maximum(acc1 + b2_ref[...], 0.0)

    pvT = _dot(w3bT_ref[...], posT_c)
    u2T = _dot(w3aT_ref[...], h1T) + pvT
    u2s = _split(u2T)
    b3c = b3_ref[...]
    w4T = w4T_ref[...]

    acc2 = _dot(w4T, jnp.maximum(u2T - pvT + b3c, 0.0))
    for t in range(1, _K):
        gs = _dot(u2s, oh_ref[t - 1])
        gT = gs[:H] + gs[H:]
        msgT = jnp.maximum(gT - pvT + b3c, 0.0)
        acc2 = jnp.maximum(acc2, _dot(w4T, msgT))
    h2T = jnp.maximum(acc2 + b4_ref[...], 0.0)

    g = jnp.max(h2T, axis=1, keepdims=True)
    out = jax.lax.dot_general(
        g, wc_ref[...], (((0,), (0,)), ((), ())),
        preferred_element_type=f32)
    out_ref[...] = (out + bc_ref[...]).reshape(out_ref.shape)


@functools.partial(jax.jit, static_argnames=())
def kernel(pos, batch, W1, b1, W2, b2, W3, b3, W4, b4, Wc, bc):
    del batch
    N = pos.shape[0]
    P = N // _B
    H = W2.shape[0]
    NCLS = Wc.shape[1]

    posT = pos.T
    w1cT = (W1[:3] + W1[3:]).T
    w1bT = W1[3:].T
    w3aT = W3[:H].T
    w3bT = W3[H:].T

    full = lambda shape: pl.BlockSpec(shape, lambda b: (0, 0))
    out = pl.pallas_call(
        _cloud_kernel,
        grid=(_B,),
        in_specs=[
            pl.BlockSpec((P, 3), lambda b: (b, 0)),
            pl.BlockSpec((3, P), lambda b: (0, b)),
            full((H, 3)),
            full((H, 3)),
            full((H, 1)),
            full((H, H)),
            full((H, 1)),
            full((H, H)),
            full((H, 3)),
            full((H, 1)),
            full((H, H)),
            full((H, 1)),
            full((H, NCLS)),
            full((1, NCLS)),
        ],
        out_specs=pl.BlockSpec((1, 1, NCLS), lambda b: (b, 0, 0)),
        out_shape=jax.ShapeDtypeStruct((_B, 1, NCLS), jnp.float32),
        scratch_shapes=[pltpu.VMEM((_K - 1, P, P), jnp.bfloat16)],
    )(
        pos, posT, w1cT, w1bT, b1.reshape(H, 1), W2.T, b2.reshape(H, 1),
        w3aT, w3bT, b3.reshape(H, 1), W4.T, b4.reshape(H, 1), Wc,
        bc.reshape(1, NCLS),
    )
    return out.reshape(_B, NCLS)
